# baseline (speedup 1.0000x reference)
import functools

import jax
import jax.numpy as jnp
from jax import lax
from jax.experimental import pallas as pl

_BIG = 3.4e38
_PEN = 1e9


def _level_body(pu_ref, bu_ref, posT_ref, bc_ref, x_ref, xs_ref,
                w_top_ref, w_bot_ref, b_up_ref, wa_ref, ba_ref,
                wb_ref, bb_ref, out_ref, *, n_coarse):
    R = pu_ref.shape[0]
    pu = pu_ref[...]
    pu2 = jnp.sum(pu * pu, axis=1, keepdims=True)
    posT = posT_ref[...]
    p2 = jnp.sum(posT * posT, axis=0, keepdims=True)
    dot = jnp.dot(pu, posT, preferred_element_type=jnp.float32,
                  precision=lax.Precision.HIGHEST)
    db = bu_ref[...] - bc_ref[...]
    d2 = jnp.maximum(pu2 - 2.0 * dot + p2, 0.0) + db * db * _PEN

    cols = lax.broadcasted_iota(jnp.int32, (R, n_coarse), 1)
    key = (lax.bitcast_convert_type(d2, jnp.int32) & ~4095) | cols
    big_key = jnp.int32(0x7F7FFFFF)

    wmat = jnp.zeros((R, n_coarse), jnp.float32)
    den = jnp.zeros((R, 1), jnp.float32)
    for _ in range(3):
        m = jnp.min(key, axis=1, keepdims=True)
        sel = key == m
        d2_k = lax.bitcast_convert_type(m & ~4095, jnp.float32)
        w = 1.0 / jnp.maximum(d2_k, 1e-16)
        wmat = jnp.where(sel, w, wmat)
        den = den + w
        key = jnp.where(sel, big_key, key)

    up = jnp.dot(wmat, x_ref[...],
                 preferred_element_type=jnp.float32) / den
    xi = jnp.dot(xs_ref[...], w_top_ref[...], preferred_element_type=jnp.float32)
    xi = xi + jnp.dot(up, w_bot_ref[...], preferred_element_type=jnp.float32)
    xi = jax.nn.relu(xi + b_up_ref[...])
    h = jax.nn.relu(jnp.dot(xi, wa_ref[...], preferred_element_type=jnp.float32)
                    + ba_ref[...])
    out_ref[...] = xi + jnp.dot(h, wb_ref[...],
                                preferred_element_type=jnp.float32) + bb_ref[...]


def _level(pos_up, batch_up, pos, batch, x, x_skip,
           W_up, b_up, Wa, ba, Wb, bb, row_tile):
    n_up = pos_up.shape[0]
    n_coarse = pos.shape[0]
    c_in = x.shape[1]
    c_skip = x_skip.shape[1]
    c_out = Wa.shape[0]
    grid = (n_up // row_tile,)

    posT = pos.T
    bu = batch_up.astype(jnp.float32)[:, None]
    bc = batch.astype(jnp.float32)[None, :]
    w_top = W_up[:c_skip]
    w_bot = W_up[c_skip:]

    row_spec = lambda cols_: pl.BlockSpec((row_tile, cols_), lambda i: (i, 0))
    full = lambda a: pl.BlockSpec(a.shape, lambda i: (0,) * a.ndim)

    fn = pl.pallas_call(
        functools.partial(_level_body, n_coarse=n_coarse),
        grid=grid,
        in_specs=[
            row_spec(3),
            row_spec(1),
            full(posT),
            full(bc),
            full(x),
            row_spec(c_skip),
            full(w_top), full(w_bot), full(b_up[None, :]),
            full(Wa), full(ba[None, :]),
            full(Wb), full(bb[None, :]),
        ],
        out_specs=row_spec(c_out),
        out_shape=jax.ShapeDtypeStruct((n_up, c_out), jnp.float32),
    )
    return fn(pos_up, bu, posT, bc, x, x_skip,
              w_top, w_bot, b_up[None, :], Wa, ba[None, :], Wb, bb[None, :])


def kernel(pos0, pos1, pos2, x0, x1, x2, batch0, batch1, batch2,
           W_up1, b_up1, W_res1a, b_res1a, W_res1b, b_res1b,
           W_up2, b_up2, W_res2a, b_res2a, W_res2b, b_res2b):
    xi1 = _level(pos1, batch1, pos0, batch0, x0, x1,
                 W_up1, b_up1, W_res1a, b_res1a, W_res1b, b_res1b,
                 row_tile=256)
    xi2 = _level(pos2, batch2, pos1, batch1, xi1, x2,
                 W_up2, b_up2, W_res2a, b_res2a, W_res2b, b_res2b,
                 row_tile=256)
    return xi2


# speedup vs baseline: 1.2819x; 1.2819x over previous
import functools

import jax
import jax.numpy as jnp
from jax import lax
from jax.experimental import pallas as pl

_BIG = 3.4e38
_PEN = 1e9


def _level_body(pu_ref, bu_ref, posT_ref, bc_ref, x_ref, xs_ref,
                w_top_ref, w_bot_ref, b_up_ref, wa_ref, ba_ref,
                wb_ref, bb_ref, out_ref, *, n_coarse):
    R = pu_ref.shape[0]
    db = bu_ref[...] - bc_ref[...]
    d2 = db * db * _PEN
    for c in range(3):
        diff = pu_ref[:, c:c + 1] - posT_ref[c:c + 1, :]
        d2 = d2 + diff * diff

    cols = lax.broadcasted_iota(jnp.int32, (R, n_coarse), 1)
    key = (lax.bitcast_convert_type(d2, jnp.int32) & ~4095) | cols
    big_key = jnp.int32(0x7F7FFFFF)

    wmat = jnp.zeros((R, n_coarse), jnp.float32)
    den = jnp.zeros((R, 1), jnp.float32)
    for _ in range(3):
        m = jnp.min(key, axis=1, keepdims=True)
        sel = key == m
        d2_k = lax.bitcast_convert_type(m & ~4095, jnp.float32)
        w = 1.0 / jnp.maximum(d2_k, 1e-16)
        wmat = jnp.where(sel, w, wmat)
        den = den + w
        key = jnp.where(sel, big_key, key)

    up = jnp.dot(wmat, x_ref[...],
                 preferred_element_type=jnp.float32) / den
    xi = jnp.dot(xs_ref[...], w_top_ref[...], preferred_element_type=jnp.float32)
    xi = xi + jnp.dot(up, w_bot_ref[...], preferred_element_type=jnp.float32)
    xi = jax.nn.relu(xi + b_up_ref[...])
    h = jax.nn.relu(jnp.dot(xi, wa_ref[...], preferred_element_type=jnp.float32)
                    + ba_ref[...])
    out_ref[...] = xi + jnp.dot(h, wb_ref[...],
                                preferred_element_type=jnp.float32) + bb_ref[...]


def _level(pos_up, batch_up, pos, batch, x, x_skip,
           W_up, b_up, Wa, ba, Wb, bb, row_tile):
    n_up = pos_up.shape[0]
    n_coarse = pos.shape[0]
    c_in = x.shape[1]
    c_skip = x_skip.shape[1]
    c_out = Wa.shape[0]
    grid = (n_up // row_tile,)

    posT = pos.T
    bu = batch_up.astype(jnp.float32)[:, None]
    bc = batch.astype(jnp.float32)[None, :]
    w_top = W_up[:c_skip]
    w_bot = W_up[c_skip:]

    row_spec = lambda cols_: pl.BlockSpec((row_tile, cols_), lambda i: (i, 0))
    full = lambda a: pl.BlockSpec(a.shape, lambda i: (0,) * a.ndim)

    fn = pl.pallas_call(
        functools.partial(_level_body, n_coarse=n_coarse),
        grid=grid,
        in_specs=[
            row_spec(3),
            row_spec(1),
            full(posT),
            full(bc),
            full(x),
            row_spec(c_skip),
            full(w_top), full(w_bot), full(b_up[None, :]),
            full(Wa), full(ba[None, :]),
            full(Wb), full(bb[None, :]),
        ],
        out_specs=row_spec(c_out),
        out_shape=jax.ShapeDtypeStruct((n_up, c_out), jnp.float32),
    )
    return fn(pos_up, bu, posT, bc, x, x_skip,
              w_top, w_bot, b_up[None, :], Wa, ba[None, :], Wb, bb[None, :])


def kernel(pos0, pos1, pos2, x0, x1, x2, batch0, batch1, batch2,
           W_up1, b_up1, W_res1a, b_res1a, W_res1b, b_res1b,
           W_up2, b_up2, W_res2a, b_res2a, W_res2b, b_res2b):
    xi1 = _level(pos1, batch1, pos0, batch0, x0, x1,
                 W_up1, b_up1, W_res1a, b_res1a, W_res1b, b_res1b,
                 row_tile=256)
    xi2 = _level(pos2, batch2, pos1, batch1, xi1, x2,
                 W_up2, b_up2, W_res2a, b_res2a, W_res2b, b_res2b,
                 row_tile=256)
    return xi2


# speedup vs baseline: 1.2873x; 1.0042x over previous
import functools

import jax
import jax.numpy as jnp
from jax import lax
from jax.experimental import pallas as pl
from jax.experimental.pallas import tpu as pltpu

_BIG_KEY = 0x7F7FFFFF
_MASK12 = ~4095


def _level_body(win_ref, pu_ref, bu_ref, posb_ref, bcb_ref, xb_ref, xs_ref,
                w_top_ref, w_bot_ref, b_up_ref, wa_ref, ba_ref,
                wb_ref, bb_ref, out_ref, m_ref, key_ref, up_ref,
                *, n_blocks, block_c):
    R = pu_ref.shape[0]
    jlo = win_ref[0, 0, 0]
    jhi = win_ref[0, 0, 1]

    m_ref[...] = jnp.full((R, 4), _BIG_KEY, jnp.int32)
    up_ref[...] = jnp.zeros_like(up_ref)

    bu = bu_ref[...]
    lane = lax.broadcasted_iota(jnp.int32, (R, block_c), 1)

    def sweep1(j, _):
        @pl.when(jnp.logical_and(j >= jlo, j < jhi))
        def _():
            d2 = jnp.zeros((R, block_c), jnp.float32)
            for c in range(3):
                diff = pu_ref[:, c:c + 1] - posb_ref[j, c:c + 1, :]
                d2 = d2 + diff * diff
            key = (lax.bitcast_convert_type(d2, jnp.int32) & _MASK12) \
                | (lane + j * block_c)
            key = jnp.where(bu == bcb_ref[j], key, _BIG_KEY)
            key_ref[j] = key
            b1 = jnp.min(key, axis=1, keepdims=True)
            key = jnp.where(key == b1, _BIG_KEY, key)
            b2 = jnp.min(key, axis=1, keepdims=True)
            key = jnp.where(key == b2, _BIG_KEY, key)
            b3 = jnp.min(key, axis=1, keepdims=True)
            a1 = m_ref[:, 0:1]
            a2 = m_ref[:, 1:2]
            a3 = m_ref[:, 2:3]
            c1 = jnp.minimum(a1, b1)
            c2 = jnp.minimum(jnp.minimum(a2, b2), jnp.maximum(a1, b1))
            c3 = jnp.minimum(jnp.minimum(a3, b3),
                             jnp.minimum(jnp.maximum(a2, b1),
                                         jnp.maximum(a1, b2)))
            m_ref[:, 0:1] = c1
            m_ref[:, 1:2] = c2
            m_ref[:, 2:3] = c3
        return 0

    lax.fori_loop(0, n_blocks, sweep1, 0, unroll=False)

    m3 = m_ref[:, 2:3]

    def sweep2(j, _):
        @pl.when(jnp.logical_and(j >= jlo, j < jhi))
        def _():
            key = key_ref[j]
            d2q = lax.bitcast_convert_type(key & _MASK12, jnp.float32)
            w = 1.0 / jnp.maximum(d2q, 1e-16)
            wmat = jnp.where(key <= m3, w, 0.0)
            up_ref[...] += jnp.dot(wmat, xb_ref[j],
                                   preferred_element_type=jnp.float32)
        return 0

    lax.fori_loop(0, n_blocks, sweep2, 0, unroll=False)

    dsel = lax.bitcast_convert_type(m_ref[:, 0:3] & _MASK12, jnp.float32)
    den = jnp.sum(1.0 / jnp.maximum(dsel, 1e-16), axis=1, keepdims=True)
    up = up_ref[...] / den

    xi = jnp.dot(xs_ref[...], w_top_ref[...], preferred_element_type=jnp.float32)
    xi = xi + jnp.dot(up, w_bot_ref[...], preferred_element_type=jnp.float32)
    xi = jax.nn.relu(xi + b_up_ref[...])
    h = jax.nn.relu(jnp.dot(xi, wa_ref[...], preferred_element_type=jnp.float32)
                    + ba_ref[...])
    out_ref[...] = xi + jnp.dot(h, wb_ref[...],
                                preferred_element_type=jnp.float32) + bb_ref[...]


def _level(pos_up, batch_up, pos, batch, x, x_skip,
           W_up, b_up, Wa, ba, Wb, bb, row_tile, block_c):
    n_up = pos_up.shape[0]
    n_coarse = pos.shape[0]
    c_skip = x_skip.shape[1]
    c_out = Wa.shape[0]
    n_blocks = n_coarse // block_c
    n_tiles = n_up // row_tile

    bu_i = batch_up.reshape(n_tiles, row_tile)
    lo = jnp.searchsorted(batch, bu_i[:, 0], side="left")
    hi = jnp.searchsorted(batch, bu_i[:, -1], side="right")
    wins = jnp.stack([lo // block_c,
                      (hi + block_c - 1) // block_c], axis=1).astype(jnp.int32)

    posb = pos.T.reshape(3, n_blocks, block_c).transpose(1, 0, 2)
    bcb = batch.astype(jnp.int32).reshape(n_blocks, 1, block_c)
    xb = x.reshape(n_blocks, block_c, x.shape[1])
    bu2 = batch_up.astype(jnp.int32)[:, None]
    w_top = W_up[:c_skip]
    w_bot = W_up[c_skip:]

    row_spec = lambda cols_: pl.BlockSpec((row_tile, cols_), lambda i: (i, 0))
    full = lambda a: pl.BlockSpec(a.shape, lambda i: (0,) * a.ndim)

    fn = pl.pallas_call(
        functools.partial(_level_body, n_blocks=n_blocks, block_c=block_c),
        grid=(n_tiles,),
        in_specs=[
            pl.BlockSpec((1, 1, 2), lambda i: (i, 0, 0),
                         memory_space=pltpu.SMEM),
            row_spec(3),
            row_spec(1),
            full(posb),
            full(bcb),
            full(xb),
            row_spec(c_skip),
            full(w_top), full(w_bot), full(b_up[None, :]),
            full(Wa), full(ba[None, :]),
            full(Wb), full(bb[None, :]),
        ],
        out_specs=row_spec(c_out),
        out_shape=jax.ShapeDtypeStruct((n_up, c_out), jnp.float32),
        scratch_shapes=[
            pltpu.VMEM((row_tile, 4), jnp.int32),
            pltpu.VMEM((n_blocks, row_tile, block_c), jnp.int32),
            pltpu.VMEM((row_tile, x.shape[1]), jnp.float32),
        ],
    )
    return fn(wins[:, None, :], pos_up, bu2, posb, bcb, xb, x_skip,
              w_top, w_bot, b_up[None, :], Wa, ba[None, :], Wb, bb[None, :])


def kernel(pos0, pos1, pos2, x0, x1, x2, batch0, batch1, batch2,
           W_up1, b_up1, W_res1a, b_res1a, W_res1b, b_res1b,
           W_up2, b_up2, W_res2a, b_res2a, W_res2b, b_res2b):
    xi1 = _level(pos1, batch1, pos0, batch0, x0, x1,
                 W_up1, b_up1, W_res1a, b_res1a, W_res1b, b_res1b,
                 row_tile=256, block_c=256)
    xi2 = _level(pos2, batch2, pos1, batch1, xi1, x2,
                 W_up2, b_up2, W_res2a, b_res2a, W_res2b, b_res2b,
                 row_tile=256, block_c=512)
    return xi2


# speedup vs baseline: 1.3098x; 1.0175x over previous
import functools

import jax
import jax.numpy as jnp
from jax import lax
from jax.experimental import pallas as pl
from jax.experimental.pallas import tpu as pltpu

_BIG_KEY = 0x7F7FFFFF
_MASK12 = ~4095


def _level_body(win_ref, pu_ref, bu_ref, posb_ref, bcb_ref, xb_ref, xs_ref,
                w_top_ref, w_bot_ref, b_up_ref, wa_ref, ba_ref,
                wb_ref, bb_ref, out_ref, m_ref, key_ref, up_ref,
                *, n_blocks, block_c):
    R = pu_ref.shape[0]
    jlo = win_ref[0, 0, 0]
    jhi = win_ref[0, 0, 1]

    m_ref[...] = jnp.full((R, 4), _BIG_KEY, jnp.int32)
    up_ref[...] = jnp.zeros_like(up_ref)

    bu = bu_ref[...]
    lane = lax.broadcasted_iota(jnp.int32, (R, block_c), 1)

    def sweep1(j, _):
        @pl.when(jnp.logical_and(j >= jlo, j < jhi))
        def _():
            d2 = jnp.zeros((R, block_c), jnp.float32)
            for c in range(3):
                diff = pu_ref[:, c:c + 1] - posb_ref[j, c:c + 1, :]
                d2 = d2 + diff * diff
            key = (lax.bitcast_convert_type(d2, jnp.int32) & _MASK12) \
                | (lane + j * block_c)
            key = jnp.where(bu == bcb_ref[j], key, _BIG_KEY)
            key_ref[j] = key
            b1 = jnp.min(key, axis=1, keepdims=True)
            key = jnp.where(key == b1, _BIG_KEY, key)
            b2 = jnp.min(key, axis=1, keepdims=True)
            key = jnp.where(key == b2, _BIG_KEY, key)
            b3 = jnp.min(key, axis=1, keepdims=True)
            a1 = m_ref[:, 0:1]
            a2 = m_ref[:, 1:2]
            a3 = m_ref[:, 2:3]
            c1 = jnp.minimum(a1, b1)
            c2 = jnp.minimum(jnp.minimum(a2, b2), jnp.maximum(a1, b1))
            c3 = jnp.minimum(jnp.minimum(a3, b3),
                             jnp.minimum(jnp.maximum(a2, b1),
                                         jnp.maximum(a1, b2)))
            m_ref[:, 0:1] = c1
            m_ref[:, 1:2] = c2
            m_ref[:, 2:3] = c3
        return 0

    lax.fori_loop(0, n_blocks, sweep1, 0, unroll=True)

    m3 = m_ref[:, 2:3]

    def sweep2(j, _):
        @pl.when(jnp.logical_and(j >= jlo, j < jhi))
        def _():
            key = key_ref[j]
            d2q = lax.bitcast_convert_type(key & _MASK12, jnp.float32)
            w = 1.0 / jnp.maximum(d2q, 1e-16)
            wmat = jnp.where(key <= m3, w, 0.0)
            up_ref[...] += jnp.dot(wmat, xb_ref[j],
                                   preferred_element_type=jnp.float32)
        return 0

    lax.fori_loop(0, n_blocks, sweep2, 0, unroll=True)

    dsel = lax.bitcast_convert_type(m_ref[:, 0:3] & _MASK12, jnp.float32)
    den = jnp.sum(1.0 / jnp.maximum(dsel, 1e-16), axis=1, keepdims=True)
    up = up_ref[...] / den

    xi = jnp.dot(xs_ref[...], w_top_ref[...], preferred_element_type=jnp.float32)
    xi = xi + jnp.dot(up, w_bot_ref[...], preferred_element_type=jnp.float32)
    xi = jax.nn.relu(xi + b_up_ref[...])
    h = jax.nn.relu(jnp.dot(xi, wa_ref[...], preferred_element_type=jnp.float32)
                    + ba_ref[...])
    out_ref[...] = xi + jnp.dot(h, wb_ref[...],
                                preferred_element_type=jnp.float32) + bb_ref[...]


def _level(pos_up, batch_up, pos, batch, x, x_skip,
           W_up, b_up, Wa, ba, Wb, bb, row_tile, block_c):
    n_up = pos_up.shape[0]
    n_coarse = pos.shape[0]
    c_skip = x_skip.shape[1]
    c_out = Wa.shape[0]
    n_blocks = n_coarse // block_c
    n_tiles = n_up // row_tile

    bu_i = batch_up.reshape(n_tiles, row_tile)
    lo = jnp.searchsorted(batch, bu_i[:, 0], side="left")
    hi = jnp.searchsorted(batch, bu_i[:, -1], side="right")
    wins = jnp.stack([lo // block_c,
                      (hi + block_c - 1) // block_c], axis=1).astype(jnp.int32)

    posb = pos.T.reshape(3, n_blocks, block_c).transpose(1, 0, 2)
    bcb = batch.astype(jnp.int32).reshape(n_blocks, 1, block_c)
    xb = x.reshape(n_blocks, block_c, x.shape[1])
    bu2 = batch_up.astype(jnp.int32)[:, None]
    w_top = W_up[:c_skip]
    w_bot = W_up[c_skip:]

    row_spec = lambda cols_: pl.BlockSpec((row_tile, cols_), lambda i: (i, 0))
    full = lambda a: pl.BlockSpec(a.shape, lambda i: (0,) * a.ndim)

    fn = pl.pallas_call(
        functools.partial(_level_body, n_blocks=n_blocks, block_c=block_c),
        grid=(n_tiles,),
        in_specs=[
            pl.BlockSpec((1, 1, 2), lambda i: (i, 0, 0),
                         memory_space=pltpu.SMEM),
            row_spec(3),
            row_spec(1),
            full(posb),
            full(bcb),
            full(xb),
            row_spec(c_skip),
            full(w_top), full(w_bot), full(b_up[None, :]),
            full(Wa), full(ba[None, :]),
            full(Wb), full(bb[None, :]),
        ],
        out_specs=row_spec(c_out),
        out_shape=jax.ShapeDtypeStruct((n_up, c_out), jnp.float32),
        scratch_shapes=[
            pltpu.VMEM((row_tile, 4), jnp.int32),
            pltpu.VMEM((n_blocks, row_tile, block_c), jnp.int32),
            pltpu.VMEM((row_tile, x.shape[1]), jnp.float32),
        ],
    )
    return fn(wins[:, None, :], pos_up, bu2, posb, bcb, xb, x_skip,
              w_top, w_bot, b_up[None, :], Wa, ba[None, :], Wb, bb[None, :])


def kernel(pos0, pos1, pos2, x0, x1, x2, batch0, batch1, batch2,
           W_up1, b_up1, W_res1a, b_res1a, W_res1b, b_res1b,
           W_up2, b_up2, W_res2a, b_res2a, W_res2b, b_res2b):
    xi1 = _level(pos1, batch1, pos0, batch0, x0, x1,
                 W_up1, b_up1, W_res1a, b_res1a, W_res1b, b_res1b,
                 row_tile=256, block_c=256)
    xi2 = _level(pos2, batch2, pos1, batch1, xi1, x2,
                 W_up2, b_up2, W_res2a, b_res2a, W_res2b, b_res2b,
                 row_tile=256, block_c=512)
    return xi2


# speedup vs baseline: 1.7315x; 1.3220x over previous
import functools

import jax
import jax.numpy as jnp
from jax import lax
from jax.experimental import pallas as pl
from jax.experimental.pallas import tpu as pltpu

_BIG_KEY = 0x7F7FFFFF
_MASK12 = ~4095


def _level_body(win_ref, pu_ref, bu_ref, posb_ref, bcb_ref, xb_ref, xs_ref,
                w_top_ref, w_bot_ref, b_up_ref, wa_ref, ba_ref,
                wb_ref, bb_ref, out_ref, m_ref, key_ref, up_ref,
                *, n_blocks, block_c):
    R = pu_ref.shape[0]
    jlo = win_ref[0, 0, 0]
    jhi = win_ref[0, 0, 1]

    LW = 128
    m_ref[...] = jnp.full((R, 3 * LW), _BIG_KEY, jnp.int32)
    up_ref[...] = jnp.zeros_like(up_ref)

    bu = bu_ref[...]
    lane = lax.broadcasted_iota(jnp.int32, (R, block_c), 1)

    def sweep1(j, _):
        @pl.when(jnp.logical_and(j >= jlo, j < jhi))
        def _():
            d2 = jnp.zeros((R, block_c), jnp.float32)
            for c in range(3):
                diff = pu_ref[:, c:c + 1] - posb_ref[j, c:c + 1, :]
                d2 = d2 + diff * diff
            key = (lax.bitcast_convert_type(d2, jnp.int32) & _MASK12) \
                | (lane + j * block_c)
            key = jnp.where(bu == bcb_ref[j], key, _BIG_KEY)
            key_ref[j] = key
            u1 = m_ref[:, 0 * LW:1 * LW]
            u2 = m_ref[:, 1 * LW:2 * LW]
            u3 = m_ref[:, 2 * LW:3 * LW]
            for s in range(block_c // LW):
                v = key[:, s * LW:(s + 1) * LW]
                t1 = jnp.maximum(u1, v)
                u1 = jnp.minimum(u1, v)
                t2 = jnp.maximum(u2, t1)
                u2 = jnp.minimum(u2, t1)
                u3 = jnp.minimum(u3, t2)
            m_ref[:, 0 * LW:1 * LW] = u1
            m_ref[:, 1 * LW:2 * LW] = u2
            m_ref[:, 2 * LW:3 * LW] = u3
        return 0

    lax.fori_loop(0, n_blocks, sweep1, 0, unroll=True)

    u1 = m_ref[:, 0 * LW:1 * LW]
    u2 = m_ref[:, 1 * LW:2 * LW]
    u3 = m_ref[:, 2 * LW:3 * LW]
    ms = []
    for _ in range(3):
        mk = jnp.min(u1, axis=1, keepdims=True)
        sel = u1 == mk
        u1 = jnp.where(sel, u2, u1)
        u2 = jnp.where(sel, u3, u2)
        u3 = jnp.where(sel, _BIG_KEY, u3)
        ms.append(mk)
    m3 = ms[2]

    def sweep2(j, _):
        @pl.when(jnp.logical_and(j >= jlo, j < jhi))
        def _():
            key = key_ref[j]
            d2q = lax.bitcast_convert_type(key & _MASK12, jnp.float32)
            w = 1.0 / jnp.maximum(d2q, 1e-16)
            wmat = jnp.where(key <= m3, w, 0.0)
            up_ref[...] += jnp.dot(wmat, xb_ref[j],
                                   preferred_element_type=jnp.float32)
        return 0

    lax.fori_loop(0, n_blocks, sweep2, 0, unroll=True)

    den = jnp.zeros((R, 1), jnp.float32)
    for mk in ms:
        dk = lax.bitcast_convert_type(mk & _MASK12, jnp.float32)
        den = den + 1.0 / jnp.maximum(dk, 1e-16)
    up = up_ref[...] / den

    xi = jnp.dot(xs_ref[...], w_top_ref[...], preferred_element_type=jnp.float32)
    xi = xi + jnp.dot(up, w_bot_ref[...], preferred_element_type=jnp.float32)
    xi = jax.nn.relu(xi + b_up_ref[...])
    h = jax.nn.relu(jnp.dot(xi, wa_ref[...], preferred_element_type=jnp.float32)
                    + ba_ref[...])
    out_ref[...] = xi + jnp.dot(h, wb_ref[...],
                                preferred_element_type=jnp.float32) + bb_ref[...]


def _level(pos_up, batch_up, pos, batch, x, x_skip,
           W_up, b_up, Wa, ba, Wb, bb, row_tile, block_c):
    n_up = pos_up.shape[0]
    n_coarse = pos.shape[0]
    c_skip = x_skip.shape[1]
    c_out = Wa.shape[0]
    n_blocks = n_coarse // block_c
    n_tiles = n_up // row_tile

    bu_i = batch_up.reshape(n_tiles, row_tile)
    lo = jnp.searchsorted(batch, bu_i[:, 0], side="left")
    hi = jnp.searchsorted(batch, bu_i[:, -1], side="right")
    wins = jnp.stack([lo // block_c,
                      (hi + block_c - 1) // block_c], axis=1).astype(jnp.int32)

    posb = pos.T.reshape(3, n_blocks, block_c).transpose(1, 0, 2)
    bcb = batch.astype(jnp.int32).reshape(n_blocks, 1, block_c)
    xb = x.reshape(n_blocks, block_c, x.shape[1])
    bu2 = batch_up.astype(jnp.int32)[:, None]
    w_top = W_up[:c_skip]
    w_bot = W_up[c_skip:]

    row_spec = lambda cols_: pl.BlockSpec((row_tile, cols_), lambda i: (i, 0))
    full = lambda a: pl.BlockSpec(a.shape, lambda i: (0,) * a.ndim)

    fn = pl.pallas_call(
        functools.partial(_level_body, n_blocks=n_blocks, block_c=block_c),
        grid=(n_tiles,),
        in_specs=[
            pl.BlockSpec((1, 1, 2), lambda i: (i, 0, 0),
                         memory_space=pltpu.SMEM),
            row_spec(3),
            row_spec(1),
            full(posb),
            full(bcb),
            full(xb),
            row_spec(c_skip),
            full(w_top), full(w_bot), full(b_up[None, :]),
            full(Wa), full(ba[None, :]),
            full(Wb), full(bb[None, :]),
        ],
        out_specs=row_spec(c_out),
        out_shape=jax.ShapeDtypeStruct((n_up, c_out), jnp.float32),
        scratch_shapes=[
            pltpu.VMEM((row_tile, 3 * 128), jnp.int32),
            pltpu.VMEM((n_blocks, row_tile, block_c), jnp.int32),
            pltpu.VMEM((row_tile, x.shape[1]), jnp.float32),
        ],
    )
    return fn(wins[:, None, :], pos_up, bu2, posb, bcb, xb, x_skip,
              w_top, w_bot, b_up[None, :], Wa, ba[None, :], Wb, bb[None, :])


def kernel(pos0, pos1, pos2, x0, x1, x2, batch0, batch1, batch2,
           W_up1, b_up1, W_res1a, b_res1a, W_res1b, b_res1b,
           W_up2, b_up2, W_res2a, b_res2a, W_res2b, b_res2b):
    xi1 = _level(pos1, batch1, pos0, batch0, x0, x1,
                 W_up1, b_up1, W_res1a, b_res1a, W_res1b, b_res1b,
                 row_tile=256, block_c=256)
    xi2 = _level(pos2, batch2, pos1, batch1, xi1, x2,
                 W_up2, b_up2, W_res2a, b_res2a, W_res2b, b_res2b,
                 row_tile=256, block_c=512)
    return xi2


# speedup vs baseline: 2.0162x; 1.1644x over previous
import functools

import jax
import jax.numpy as jnp
from jax import lax
from jax.experimental import pallas as pl
from jax.experimental.pallas import tpu as pltpu

_BIG_KEY = 0x7F7FFFFF
_MASK12 = ~4095


def _level_body(win_ref, pu_ref, bu_ref, posb_ref, bcb_ref, xb_ref, xs_ref,
                w_top_ref, w_bot_ref, b_up_ref, wa_ref, ba_ref,
                wb_ref, bb_ref, out_ref, m_ref, key_ref, up_ref,
                *, n_blocks, block_c):
    R = pu_ref.shape[0]
    jlo = win_ref[0, 0, 0]
    jhi = win_ref[0, 0, 1]

    LW = 128
    m_ref[...] = jnp.full((R, 3 * LW), _BIG_KEY, jnp.int32)
    up_ref[...] = jnp.zeros_like(up_ref)

    bu = bu_ref[...]
    lane = lax.broadcasted_iota(jnp.int32, (R, block_c), 1)

    def sweep1(j, _):
        @pl.when(jnp.logical_and(j >= jlo, j < jhi))
        def _():
            d2 = jnp.zeros((R, block_c), jnp.float32)
            for c in range(3):
                diff = pu_ref[:, c:c + 1] - posb_ref[j, c:c + 1, :]
                d2 = d2 + diff * diff
            key = (lax.bitcast_convert_type(d2, jnp.int32) & _MASK12) \
                | (lane + j * block_c)
            key = jnp.where(bu == bcb_ref[j], key, _BIG_KEY)
            key_ref[j] = key
            u1 = m_ref[:, 0 * LW:1 * LW]
            u2 = m_ref[:, 1 * LW:2 * LW]
            u3 = m_ref[:, 2 * LW:3 * LW]
            for s in range(block_c // LW):
                v = key[:, s * LW:(s + 1) * LW]
                t1 = jnp.maximum(u1, v)
                u1 = jnp.minimum(u1, v)
                t2 = jnp.maximum(u2, t1)
                u2 = jnp.minimum(u2, t1)
                u3 = jnp.minimum(u3, t2)
            m_ref[:, 0 * LW:1 * LW] = u1
            m_ref[:, 1 * LW:2 * LW] = u2
            m_ref[:, 2 * LW:3 * LW] = u3
        return 0

    lax.fori_loop(0, n_blocks, sweep1, 0, unroll=True)

    u1 = m_ref[:, 0 * LW:1 * LW]
    u2 = m_ref[:, 1 * LW:2 * LW]
    u3 = m_ref[:, 2 * LW:3 * LW]
    ms = []
    for _ in range(3):
        mk = jnp.min(u1, axis=1, keepdims=True)
        sel = u1 == mk
        u1 = jnp.where(sel, u2, u1)
        u2 = jnp.where(sel, u3, u2)
        u3 = jnp.where(sel, _BIG_KEY, u3)
        ms.append(mk)
    m3 = ms[2]

    def sweep2(j, _):
        @pl.when(jnp.logical_and(j >= jlo, j < jhi))
        def _():
            key = key_ref[j]
            d2q = lax.bitcast_convert_type(key & _MASK12, jnp.float32)
            w = 1.0 / jnp.maximum(d2q, 1e-16)
            wmat = jnp.where(key <= m3, w, 0.0)
            up_ref[...] += jnp.dot(wmat, xb_ref[j],
                                   preferred_element_type=jnp.float32)
        return 0

    lax.fori_loop(0, n_blocks, sweep2, 0, unroll=True)

    den = jnp.zeros((R, 1), jnp.float32)
    for mk in ms:
        dk = lax.bitcast_convert_type(mk & _MASK12, jnp.float32)
        den = den + 1.0 / jnp.maximum(dk, 1e-16)
    up = up_ref[...] / den

    xi = jnp.dot(xs_ref[...], w_top_ref[...], preferred_element_type=jnp.float32)
    xi = xi + jnp.dot(up, w_bot_ref[...], preferred_element_type=jnp.float32)
    xi = jax.nn.relu(xi + b_up_ref[...])
    h = jax.nn.relu(jnp.dot(xi, wa_ref[...], preferred_element_type=jnp.float32)
                    + ba_ref[...])
    out_ref[...] = xi + jnp.dot(h, wb_ref[...],
                                preferred_element_type=jnp.float32) + bb_ref[...]


def _level(pos_up, batch_up, pos, batch, x, x_skip,
           W_up, b_up, Wa, ba, Wb, bb, row_tile, block_c):
    n_up = pos_up.shape[0]
    n_coarse = pos.shape[0]
    c_skip = x_skip.shape[1]
    c_out = Wa.shape[0]
    n_blocks = n_coarse // block_c
    n_tiles = n_up // row_tile

    bu_i = batch_up.reshape(n_tiles, row_tile)
    lo = jnp.searchsorted(batch, bu_i[:, 0], side="left")
    hi = jnp.searchsorted(batch, bu_i[:, -1], side="right")
    wins = jnp.stack([lo // block_c,
                      (hi + block_c - 1) // block_c], axis=1).astype(jnp.int32)

    posb = pos.T.reshape(3, n_blocks, block_c).transpose(1, 0, 2)
    bcb = batch.astype(jnp.int32).reshape(n_blocks, 1, block_c)
    xb = x.reshape(n_blocks, block_c, x.shape[1])
    bu2 = batch_up.astype(jnp.int32)[:, None]
    w_top = W_up[:c_skip]
    w_bot = W_up[c_skip:]

    row_spec = lambda cols_: pl.BlockSpec((row_tile, cols_), lambda i: (i, 0))
    full = lambda a: pl.BlockSpec(a.shape, lambda i: (0,) * a.ndim)

    fn = pl.pallas_call(
        functools.partial(_level_body, n_blocks=n_blocks, block_c=block_c),
        grid=(n_tiles,),
        in_specs=[
            pl.BlockSpec((1, 1, 2), lambda i: (i, 0, 0),
                         memory_space=pltpu.SMEM),
            row_spec(3),
            row_spec(1),
            full(posb),
            full(bcb),
            full(xb),
            row_spec(c_skip),
            full(w_top), full(w_bot), full(b_up[None, :]),
            full(Wa), full(ba[None, :]),
            full(Wb), full(bb[None, :]),
        ],
        out_specs=row_spec(c_out),
        out_shape=jax.ShapeDtypeStruct((n_up, c_out), jnp.float32),
        scratch_shapes=[
            pltpu.VMEM((row_tile, 3 * 128), jnp.int32),
            pltpu.VMEM((n_blocks, row_tile, block_c), jnp.int32),
            pltpu.VMEM((row_tile, x.shape[1]), jnp.float32),
        ],
    )
    return fn(wins[:, None, :], pos_up, bu2, posb, bcb, xb, x_skip,
              w_top, w_bot, b_up[None, :], Wa, ba[None, :], Wb, bb[None, :])


def kernel(pos0, pos1, pos2, x0, x1, x2, batch0, batch1, batch2,
           W_up1, b_up1, W_res1a, b_res1a, W_res1b, b_res1b,
           W_up2, b_up2, W_res2a, b_res2a, W_res2b, b_res2b):
    xi1 = _level(pos1, batch1, pos0, batch0, x0, x1,
                 W_up1, b_up1, W_res1a, b_res1a, W_res1b, b_res1b,
                 row_tile=512, block_c=256)
    xi2 = _level(pos2, batch2, pos1, batch1, xi1, x2,
                 W_up2, b_up2, W_res2a, b_res2a, W_res2b, b_res2b,
                 row_tile=512, block_c=256)
    return xi2
